# baseline (speedup 1.0000x reference)
import functools

import jax
import jax.numpy as jnp
from jax import lax
from jax.experimental import pallas as pl
from jax.experimental.pallas import tpu as pltpu
from jax.experimental.pallas import tpu_sc as plsc

B = 4096
D = 64
HIST = 50
HIST_P = 56
H1, H2 = 64, 32
NC, NS, L = 2, 16, 16
NW = NC * NS
BPW = B // NW
NCH = D // L

RPC = 4
CIDX = RPC * HIST_P
NCHUNK = 2 * BPW // RPC
NBUF = 4

_mesh = plsc.VectorSubcoreMesh(
    core_axis_name="c", subcore_axis_name="s", num_cores=NC, num_subcores=NS)


@functools.partial(
    pl.kernel,
    out_type=[jax.ShapeDtypeStruct((B, D), jnp.float32)] * 6,
    mesh=_mesh,
    scratch_types=[
        pltpu.VMEM((NCHUNK * CIDX,), jnp.int32),
        pltpu.VMEM((NBUF, CIDX, D), jnp.float32),
        pltpu.VMEM((2 * BPW, D), jnp.float32),
        pltpu.VMEM((4, BPW), jnp.int32),
        pltpu.VMEM((4, BPW, D), jnp.float32),
        pltpu.SemaphoreType.DMA,
        pltpu.SemaphoreType.DMA,
        pltpu.SemaphoreType.DMA,
        pltpu.SemaphoreType.DMA,
        pltpu.SemaphoreType.DMA,
    ],
    compiler_params=pltpu.CompilerParams(use_tc_tiling_on_sc=False),
)
def _sc_embed(uidx, aidx, utidx, iidx, cidx, itidx,
              uid_tab, uage_tab, text_tab, iid_tab, icate_tab,
              out_uid, out_uage, out_utx, out_iid, out_icate, out_itx,
              tidx_v, bufs_v, pool_v, fidx_v, frows_v,
              sem0, sem1, sem2, sem3, semf):
    sems = (sem0, sem1, sem2, sem3)
    wid = lax.axis_index("s") * NC + lax.axis_index("c")
    base = wid * BPW

    field_in = ((uidx, uid_tab), (aidx, uage_tab),
                (iidx, iid_tab), (cidx, icate_tab))
    for f, (idx_hbm, tab) in enumerate(field_in):
        pltpu.sync_copy(idx_hbm.at[pl.ds(base, BPW)], fidx_v.at[f])
        pltpu.async_copy(tab.at[fidx_v.at[f]], frows_v.at[f], semf)

    half = NCHUNK // 2 * CIDX
    pltpu.sync_copy(
        utidx.at[pl.ds(base * HIST_P, BPW * HIST_P)],
        tidx_v.at[pl.ds(0, half)])
    pltpu.sync_copy(
        itidx.at[pl.ds(base * HIST_P, BPW * HIST_P)],
        tidx_v.at[pl.ds(half, half)])

    def fire(k, n):
        pltpu.async_copy(
            text_tab.at[tidx_v.at[pl.ds(k * CIDX, CIDX)]],
            bufs_v.at[n], sems[n])

    def wait(k, n):
        pltpu.make_async_copy(
            text_tab.at[tidx_v.at[pl.ds(k * CIDX, CIDX)]],
            bufs_v.at[n], sems[n]).wait()

    def accumulate(k, n):
        for bb in range(RPC):
            r = bufs_v.at[n].at[pl.ds(bb * HIST_P, HIST_P)]
            zero = jnp.zeros((L,), jnp.float32)

            def inner(j, accs):
                return tuple(
                    accs[c] + r[j, pl.ds(c * L, L)] for c in range(NCH))

            accs = lax.fori_loop(0, HIST, inner, (zero,) * NCH, unroll=2)
            row = k * RPC + bb
            for c in range(NCH):
                pool_v[row, pl.ds(c * L, L)] = accs[c] * (1.0 / HIST)

    for n in range(NBUF):
        fire(n, n)

    def body(kk, carry):
        for n in range(NBUF):
            k = kk * NBUF + n
            wait(k, n)
            accumulate(k, n)

            @pl.when(k + NBUF < NCHUNK)
            def _():
                fire(k + NBUF, n)
        return carry

    lax.fori_loop(0, NCHUNK // NBUF, body, 0)

    pltpu.sync_copy(pool_v.at[pl.ds(0, BPW)], out_utx.at[pl.ds(base, BPW), :])
    pltpu.sync_copy(pool_v.at[pl.ds(BPW, BPW)],
                    out_itx.at[pl.ds(base, BPW), :])

    field_out = ((uid_tab, out_uid), (uage_tab, out_uage),
                 (iid_tab, out_iid), (icate_tab, out_icate))
    for f, (tab, out_hbm) in enumerate(field_out):
        pltpu.make_async_copy(
            tab.at[fidx_v.at[f]], frows_v.at[f], semf).wait()
        pltpu.sync_copy(frows_v.at[f], out_hbm.at[pl.ds(base, BPW), :])


def _tc_body(euid, euage, eutx, eiid, eicate, eitx,
             uw1, ub1, uw2, ub2, iw1, ib1, iw2, ib2, out):
    f32 = jnp.float32

    def tower(e1, e2, e3, w1, b1, w2, b2):
        h = (jnp.dot(e1[...], w1[0:D], preferred_element_type=f32)
             + jnp.dot(e2[...], w1[D:2 * D], preferred_element_type=f32)
             + jnp.dot(e3[...], w1[2 * D:3 * D], preferred_element_type=f32)
             + b1[...])
        h = jnp.maximum(h, 0.0)
        o = jnp.dot(h, w2[...], preferred_element_type=f32) + b2[...]
        return jnp.maximum(o, 0.0)

    u = tower(euid, euage, eutx, uw1, ub1, uw2, ub2)
    it = tower(eiid, eicate, eitx, iw1, ib1, iw2, ib2)
    dot = jnp.sum(u * it)
    nu = jnp.sum(u * u)
    ni = jnp.sum(it * it)
    out[0, 0] = dot / (jnp.sqrt(nu) * jnp.sqrt(ni))


_tc_call = pl.pallas_call(
    _tc_body,
    out_shape=jax.ShapeDtypeStruct((1, 1), jnp.float32),
    out_specs=pl.BlockSpec(memory_space=pltpu.SMEM),
)


def kernel(user_id, user_age, user_text, item_id, item_cate, item_text,
           user_id_table, user_age_table, text_table, item_id_table,
           item_cate_table, u_w1, u_b1, u_w2, u_b2, i_w1, i_b1, i_w2, i_b2):
    uidx = user_id.reshape(B).astype(jnp.int32)
    aidx = user_age.reshape(B).astype(jnp.int32)
    iidx = item_id.reshape(B).astype(jnp.int32)
    cidx = item_cate.reshape(B).astype(jnp.int32)
    utp = jnp.pad(user_text.astype(jnp.int32),
                  ((0, 0), (0, HIST_P - HIST))).reshape(B * HIST_P)
    itp = jnp.pad(item_text.astype(jnp.int32),
                  ((0, 0), (0, HIST_P - HIST))).reshape(B * HIST_P)
    euid, euage, eutx, eiid, eicate, eitx = _sc_embed(
        uidx, aidx, utp, iidx, cidx, itp,
        user_id_table, user_age_table, text_table, item_id_table,
        item_cate_table)
    score = _tc_call(
        euid, euage, eutx, eiid, eicate, eitx,
        u_w1, u_b1.reshape(1, H1), u_w2, u_b2.reshape(1, H2),
        i_w1, i_b1.reshape(1, H1), i_w2, i_b2.reshape(1, H2))
    return score.reshape(())


# speedup vs baseline: 1.5132x; 1.5132x over previous
import functools

import jax
import jax.numpy as jnp
import numpy as np
from jax import lax
from jax.experimental import pallas as pl
from jax.experimental.pallas import tpu as pltpu
from jax.experimental.pallas import tpu_sc as plsc

B = 4096
D = 64
HIST = 50
HIST_P = 56
H1, H2 = 64, 32
NC, NS, L = 2, 16, 16
NW = NC * NS
BPW = B // NW
NCH = D // L

RPC = 4
CIDX = RPC * HIST_P
NCHUNK = 2 * BPW // RPC
NBUF = 4

_UNPACK_PERM = np.zeros((D,), dtype=np.int32)
for _c in range(D // 32):
    for _j in range(16):
        _UNPACK_PERM[32 * _c + 2 * _j] = 32 * _c + _j
        _UNPACK_PERM[32 * _c + 2 * _j + 1] = 32 * _c + 16 + _j

_mesh = plsc.VectorSubcoreMesh(
    core_axis_name="c", subcore_axis_name="s", num_cores=NC, num_subcores=NS)


@functools.partial(
    pl.kernel,
    out_type=[jax.ShapeDtypeStruct((B, D), jnp.float32)] * 6,
    mesh=_mesh,
    scratch_types=[
        pltpu.VMEM((NCHUNK * CIDX,), jnp.int32),
        pltpu.VMEM((NBUF, CIDX, D), jnp.bfloat16),
        pltpu.VMEM((2 * BPW, D), jnp.float32),
        pltpu.VMEM((4, BPW), jnp.int32),
        pltpu.VMEM((4, BPW, D), jnp.float32),
        pltpu.SemaphoreType.DMA,
        pltpu.SemaphoreType.DMA,
        pltpu.SemaphoreType.DMA,
        pltpu.SemaphoreType.DMA,
        pltpu.SemaphoreType.DMA,
    ],
    compiler_params=pltpu.CompilerParams(
        use_tc_tiling_on_sc=False, needs_layout_passes=False),
)
def _sc_embed(uidx, aidx, utidx, iidx, cidx, itidx,
              uid_tab, uage_tab, text_tab, iid_tab, icate_tab,
              out_uid, out_uage, out_utx, out_iid, out_icate, out_itx,
              tidx_v, bufs_v, pool_v, fidx_v, frows_v,
              sem0, sem1, sem2, sem3, semf):
    sems = (sem0, sem1, sem2, sem3)
    wid = lax.axis_index("s") * NC + lax.axis_index("c")
    base = wid * BPW

    field_in = ((uidx, uid_tab), (aidx, uage_tab),
                (iidx, iid_tab), (cidx, icate_tab))
    for f, (idx_hbm, tab) in enumerate(field_in):
        pltpu.sync_copy(idx_hbm.at[pl.ds(base, BPW)], fidx_v.at[f])
        pltpu.async_copy(tab.at[fidx_v.at[f]], frows_v.at[f], semf)

    half = NCHUNK // 2 * CIDX
    pltpu.sync_copy(
        utidx.at[pl.ds(base * HIST_P, BPW * HIST_P)],
        tidx_v.at[pl.ds(0, half)])
    pltpu.sync_copy(
        itidx.at[pl.ds(base * HIST_P, BPW * HIST_P)],
        tidx_v.at[pl.ds(half, half)])

    def fire(k, n):
        pltpu.async_copy(
            text_tab.at[tidx_v.at[pl.ds(k * CIDX, CIDX)]],
            bufs_v.at[n], sems[n])

    def wait(k, n):
        pltpu.make_async_copy(
            text_tab.at[tidx_v.at[pl.ds(k * CIDX, CIDX)]],
            bufs_v.at[n], sems[n]).wait()

    def accumulate(k, n):
        for bb in range(RPC):
            r = bufs_v.at[n].at[pl.ds(bb * HIST_P, HIST_P)]
            zero = jnp.zeros((L,), jnp.float32)

            def inner(j, accs):
                out = []
                for h in range(NCH // 2):
                    lo, hi = plsc.unpack(
                        r[j, pl.ds(h * 2 * L, 2 * L)],
                        format=plsc.PackFormat.INTERLEAVED)
                    out.append(accs[2 * h] + lo)
                    out.append(accs[2 * h + 1] + hi)
                return tuple(out)

            accs = lax.fori_loop(0, HIST, inner, (zero,) * NCH, unroll=2)
            row = k * RPC + bb
            for c in range(NCH):
                pool_v[row, pl.ds(c * L, L)] = accs[c] * (1.0 / HIST)

    for n in range(NBUF):
        fire(n, n)

    def body(kk, carry):
        for n in range(NBUF):
            k = kk * NBUF + n
            wait(k, n)
            accumulate(k, n)

            @pl.when(k + NBUF < NCHUNK)
            def _():
                fire(k + NBUF, n)
        return carry

    lax.fori_loop(0, NCHUNK // NBUF, body, 0)

    pltpu.sync_copy(pool_v.at[pl.ds(0, BPW)], out_utx.at[pl.ds(base, BPW), :])
    pltpu.sync_copy(pool_v.at[pl.ds(BPW, BPW)],
                    out_itx.at[pl.ds(base, BPW), :])

    field_out = ((uid_tab, out_uid), (uage_tab, out_uage),
                 (iid_tab, out_iid), (icate_tab, out_icate))
    for f, (tab, out_hbm) in enumerate(field_out):
        pltpu.make_async_copy(
            tab.at[fidx_v.at[f]], frows_v.at[f], semf).wait()
        pltpu.sync_copy(frows_v.at[f], out_hbm.at[pl.ds(base, BPW), :])


def _tc_body(euid, euage, eutx, eiid, eicate, eitx,
             uw1, ub1, uw2, ub2, iw1, ib1, iw2, ib2, out):
    f32 = jnp.float32

    def tower(e1, e2, e3, w1, b1, w2, b2):
        h = (jnp.dot(e1[...], w1[0:D], preferred_element_type=f32)
             + jnp.dot(e2[...], w1[D:2 * D], preferred_element_type=f32)
             + jnp.dot(e3[...], w1[2 * D:3 * D], preferred_element_type=f32)
             + b1[...])
        h = jnp.maximum(h, 0.0)
        o = jnp.dot(h, w2[...], preferred_element_type=f32) + b2[...]
        return jnp.maximum(o, 0.0)

    u = tower(euid, euage, eutx, uw1, ub1, uw2, ub2)
    it = tower(eiid, eicate, eitx, iw1, ib1, iw2, ib2)
    dot = jnp.sum(u * it)
    nu = jnp.sum(u * u)
    ni = jnp.sum(it * it)
    out[0, 0] = dot / (jnp.sqrt(nu) * jnp.sqrt(ni))


_tc_call = pl.pallas_call(
    _tc_body,
    out_shape=jax.ShapeDtypeStruct((1, 1), jnp.float32),
    out_specs=pl.BlockSpec(memory_space=pltpu.SMEM),
)


def kernel(user_id, user_age, user_text, item_id, item_cate, item_text,
           user_id_table, user_age_table, text_table, item_id_table,
           item_cate_table, u_w1, u_b1, u_w2, u_b2, i_w1, i_b1, i_w2, i_b2):
    uidx = user_id.reshape(B).astype(jnp.int32)
    aidx = user_age.reshape(B).astype(jnp.int32)
    iidx = item_id.reshape(B).astype(jnp.int32)
    cidx = item_cate.reshape(B).astype(jnp.int32)
    utp = jnp.pad(user_text.astype(jnp.int32),
                  ((0, 0), (0, HIST_P - HIST))).reshape(B * HIST_P)
    itp = jnp.pad(item_text.astype(jnp.int32),
                  ((0, 0), (0, HIST_P - HIST))).reshape(B * HIST_P)
    text_table = text_table.astype(jnp.bfloat16)[:, _UNPACK_PERM]
    euid, euage, eutx, eiid, eicate, eitx = _sc_embed(
        uidx, aidx, utp, iidx, cidx, itp,
        user_id_table, user_age_table, text_table, item_id_table,
        item_cate_table)
    score = _tc_call(
        euid, euage, eutx, eiid, eicate, eitx,
        u_w1, u_b1.reshape(1, H1), u_w2, u_b2.reshape(1, H2),
        i_w1, i_b1.reshape(1, H1), i_w2, i_b2.reshape(1, H2))
    return score.reshape(())


# speedup vs baseline: 1.7148x; 1.1333x over previous
import functools

import jax
import jax.numpy as jnp
import numpy as np
from jax import lax
from jax.experimental import pallas as pl
from jax.experimental.pallas import tpu as pltpu
from jax.experimental.pallas import tpu_sc as plsc

B = 4096
D = 64
HIST = 50
HIST_P = 64
H1, H2 = 64, 32
NC, NS, L = 2, 16, 16
NW = NC * NS
BPW = B // NW
NCH = D // L
DP = 2 * D

VT = 100001
QL = 25088
QP = QL // 2
VT_PAD = 4 * QL
STAGE_PER_TILE = QP // NS

SECT = 128
NSEC = 2
IDX_PER_SECT = SECT * HIST_P
MAXC = SECT * HIST
CCAP = MAXC + 256
DUMMY = SECT
SENT = 1 << 20

CIDX = 128
NBUF = 2

_UNPACK_PERM = np.zeros((D,), dtype=np.int32)
for _c in range(D // 32):
    for _j in range(16):
        _UNPACK_PERM[32 * _c + 2 * _j] = 32 * _c + _j
        _UNPACK_PERM[32 * _c + 2 * _j + 1] = 32 * _c + 16 + _j

_mesh = plsc.VectorSubcoreMesh(
    core_axis_name="c", subcore_axis_name="s", num_cores=NC, num_subcores=NS)


@functools.partial(
    pl.kernel,
    out_type=[jax.ShapeDtypeStruct((B, D), jnp.float32)] * 12,
    mesh=_mesh,
    scratch_types=[
        pltpu.VMEM((IDX_PER_SECT,), jnp.int32),
        pltpu.VMEM((CCAP,), jnp.int32),
        pltpu.VMEM((CCAP,), jnp.int32),
        pltpu.VMEM((NBUF, CIDX, DP), jnp.bfloat16),
        pltpu.VMEM((DUMMY + 8, D), jnp.float32),
        pltpu.VMEM((4, BPW), jnp.int32),
        pltpu.VMEM((2, BPW, D), jnp.float32),
        pltpu.VMEM_SHARED((QP, DP), jnp.bfloat16),
        pltpu.SemaphoreType.DMA,
        pltpu.SemaphoreType.DMA,
        pltpu.SemaphoreType.DMA,
    ],
    compiler_params=pltpu.CompilerParams(
        use_tc_tiling_on_sc=False, needs_layout_passes=False),
)
def _sc_embed(uidx, aidx, utidx, iidx, cidx_in, itidx,
              uid_tab, uage_tab, text_tab, iid_tab, icate_tab,
              out_uid, out_uage, out_iid, out_icate,
              out_utx0, out_utx1, out_utx2, out_utx3,
              out_itx0, out_itx1, out_itx2, out_itx3,
              idxraw_v, cidx_v, cslot_v, bufs_v, pool_v, fidx_v, frows_v,
              shared_tab, sem0, sem1, semf):
    sems = (sem0, sem1)
    c = lax.axis_index("c")
    s = lax.axis_index("s")
    wid = s * NC + c
    fbase = wid * BPW

    field_in = ((uidx, uid_tab), (aidx, uage_tab),
                (iidx, iid_tab), (cidx_in, icate_tab))
    field_out = (out_uid, out_uage, out_iid, out_icate)

    def field_fire(f):
        pltpu.sync_copy(field_in[f][0].at[pl.ds(fbase, BPW)], fidx_v.at[f])
        pltpu.async_copy(
            field_in[f][1].at[fidx_v.at[f]], frows_v.at[f % 2], semf)

    def field_drain(f):
        pltpu.make_async_copy(
            field_in[f][1].at[fidx_v.at[f]], frows_v.at[f % 2], semf).wait()
        pltpu.sync_copy(
            frows_v.at[f % 2], field_out[f].at[pl.ds(fbase, BPW), :])

    field_fire(0)
    field_fire(1)

    tower_in = (utidx, itidx)
    tower_out = ((out_utx0, out_utx1, out_utx2, out_utx3),
                 (out_itx0, out_itx1, out_itx2, out_itx3))

    def section(qbase, tidx_hbm, out0, out1, hb):
        def zrow(i, carry):
            for cc in range(NCH):
                pool_v[i, pl.ds(cc * L, L)] = jnp.zeros((L,), jnp.float32)
            return carry

        lax.fori_loop(0, DUMMY + 1, zrow, 0)

        pltpu.sync_copy(
            tidx_hbm.at[pl.ds(s * 2 * IDX_PER_SECT + hb * IDX_PER_SECT,
                              IDX_PER_SECT)], idxraw_v)

        def grp(g, off):
            for cc in range(HIST_P // L):
                v = idxraw_v[pl.ds((g * (HIST_P // L) + cc) * L, L)]
                m = (v >= qbase) & (v < qbase + QL)
                lv = v - qbase
                plsc.store_compressed(
                    cidx_v.at[pl.ds(off, L)],
                    lax.shift_right_logical(lv, 1), mask=m)
                plsc.store_compressed(
                    cslot_v.at[pl.ds(off, L)],
                    jnp.full((L,), 2 * g, jnp.int32) + (lv & 1), mask=m)
                off = off + plsc.all_reduce_population_count(m)[0]
            return off

        total = lax.fori_loop(0, SECT, grp, 0)

        def fill(i, carry):
            cidx_v[pl.ds(total + i * L, L)] = jnp.zeros((L,), jnp.int32)
            cslot_v[pl.ds(total + i * L, L)] = jnp.full(
                (L,), 2 * DUMMY, jnp.int32)
            return carry

        lax.fori_loop(0, CIDX // L + 1, fill, 0)
        nchunk = (total + CIDX - 1) // CIDX

        def fire(k, n):
            pltpu.async_copy(
                shared_tab.at[cidx_v.at[pl.ds(k * CIDX, CIDX)]],
                bufs_v.at[n], sems[n])

        def wait(k, n):
            pltpu.make_async_copy(
                shared_tab.at[cidx_v.at[pl.ds(k * CIDX, CIDX)]],
                bufs_v.at[n], sems[n]).wait()

        def accumulate(k, n):
            rb = bufs_v.at[n]

            def row(r, carry):
                enc = cslot_v[pl.ds(k * CIDX + r, L)][0]
                slot = lax.shift_right_logical(enc, 1)
                colbase = (enc & 1) * D
                for h in range(NCH // 2):
                    lo, hi = plsc.unpack(
                        rb[r, pl.ds(colbase + h * 2 * L, 2 * L)],
                        format=plsc.PackFormat.INTERLEAVED)
                    plsc.addupdate(
                        pool_v.at[slot, pl.ds(2 * h * L, L)], lo)
                    plsc.addupdate(
                        pool_v.at[slot, pl.ds((2 * h + 1) * L, L)], hi)
                return carry

            lax.fori_loop(0, CIDX, row, 0)

        for n in range(NBUF):
            @pl.when(n < nchunk)
            def _():
                fire(n, n)

        def body(kk, carry):
            for n in range(NBUF):
                k = kk * NBUF + n

                @pl.when(k < nchunk)
                def _():
                    wait(k, n)
                    accumulate(k, n)

                @pl.when(k + NBUF < nchunk)
                def _():
                    fire(k + NBUF, n)
            return carry

        lax.fori_loop(0, (nchunk + NBUF - 1) // NBUF, body, 0)

        rowbase = s * 2 * SECT + hb * SECT

        @pl.when(c == 0)
        def _():
            pltpu.sync_copy(pool_v.at[pl.ds(0, SECT)],
                            out0.at[pl.ds(rowbase, SECT), :])

        @pl.when(c == 1)
        def _():
            pltpu.sync_copy(pool_v.at[pl.ds(0, SECT)],
                            out1.at[pl.ds(rowbase, SECT), :])

    for pss in range(2):
        plsc.subcore_barrier()
        qbase = (2 * pss) * QL + c * QL
        pltpu.sync_copy(
            text_tab.at[pl.ds(pss * 2 * QP + c * QP + s * STAGE_PER_TILE,
                              STAGE_PER_TILE), :],
            shared_tab.at[pl.ds(s * STAGE_PER_TILE, STAGE_PER_TILE), :])
        plsc.subcore_barrier()

        for tw in range(2):
            outs = tower_out[tw]
            for hb in range(2):
                section(qbase, tower_in[tw],
                        outs[2 * pss], outs[2 * pss + 1], hb)

        if pss == 0:
            field_drain(0)
            field_fire(2)
            field_drain(1)
            field_fire(3)

    field_drain(2)
    field_drain(3)


def _tc_body(euid, euage, eutx0, eutx1, eutx2, eutx3,
             eiid, eicate, eitx0, eitx1, eitx2, eitx3,
             uw1, ub1, uw2, ub2, iw1, ib1, iw2, ib2, out):
    f32 = jnp.float32

    def tower(e1, e2, e3, w1, b1, w2, b2):
        h = (jnp.dot(e1, w1[0:D], preferred_element_type=f32)
             + jnp.dot(e2, w1[D:2 * D], preferred_element_type=f32)
             + jnp.dot(e3, w1[2 * D:3 * D], preferred_element_type=f32)
             + b1[...])
        h = jnp.maximum(h, 0.0)
        o = jnp.dot(h, w2[...], preferred_element_type=f32) + b2[...]
        return jnp.maximum(o, 0.0)

    eutx = eutx0[...] + eutx1[...] + eutx2[...] + eutx3[...]
    eitx = eitx0[...] + eitx1[...] + eitx2[...] + eitx3[...]
    u = tower(euid[...], euage[...], eutx, uw1, ub1, uw2, ub2)
    it = tower(eiid[...], eicate[...], eitx, iw1, ib1, iw2, ib2)
    dot = jnp.sum(u * it)
    nu = jnp.sum(u * u)
    ni = jnp.sum(it * it)
    out[0, 0] = dot / (jnp.sqrt(nu) * jnp.sqrt(ni))


_tc_call = pl.pallas_call(
    _tc_body,
    out_shape=jax.ShapeDtypeStruct((1, 1), jnp.float32),
    out_specs=pl.BlockSpec(memory_space=pltpu.SMEM),
)


def kernel(user_id, user_age, user_text, item_id, item_cate, item_text,
           user_id_table, user_age_table, text_table, item_id_table,
           item_cate_table, u_w1, u_b1, u_w2, u_b2, i_w1, i_b1, i_w2, i_b2):
    uidx = user_id.reshape(B).astype(jnp.int32)
    aidx = user_age.reshape(B).astype(jnp.int32)
    iidx = item_id.reshape(B).astype(jnp.int32)
    cidx = item_cate.reshape(B).astype(jnp.int32)
    utp = jnp.pad(user_text.astype(jnp.int32),
                  ((0, 0), (0, HIST_P - HIST)),
                  constant_values=SENT).reshape(B * HIST_P)
    itp = jnp.pad(item_text.astype(jnp.int32),
                  ((0, 0), (0, HIST_P - HIST)),
                  constant_values=SENT).reshape(B * HIST_P)
    tt = jnp.pad((text_table * (1.0 / HIST)).astype(jnp.bfloat16)
                 [:, _UNPACK_PERM],
                 ((0, VT_PAD - VT), (0, 0))).reshape(VT_PAD // 2, DP)
    (euid, euage, eiid, eicate,
     eutx0, eutx1, eutx2, eutx3,
     eitx0, eitx1, eitx2, eitx3) = _sc_embed(
        uidx, aidx, utp, iidx, cidx, itp,
        user_id_table, user_age_table, tt, item_id_table, item_cate_table)
    score = _tc_call(
        euid, euage, eutx0, eutx1, eutx2, eutx3,
        eiid, eicate, eitx0, eitx1, eitx2, eitx3,
        u_w1, u_b1.reshape(1, H1), u_w2, u_b2.reshape(1, H2),
        i_w1, i_b1.reshape(1, H1), i_w2, i_b2.reshape(1, H2))
    return score.reshape(())


# speedup vs baseline: 1.7197x; 1.0028x over previous
import functools

import jax
import jax.numpy as jnp
import numpy as np
from jax import lax
from jax.experimental import pallas as pl
from jax.experimental.pallas import tpu as pltpu
from jax.experimental.pallas import tpu_sc as plsc

B = 4096
D = 64
HIST = 50
HIST_P = 64
H1, H2 = 64, 32
NC, NS, L = 2, 16, 16
NW = NC * NS
BPW = B // NW
NCH = D // L
DP = 2 * D

VT = 100001
QL = 25088
QP = QL // 2
VT_PAD = 4 * QL
STAGE_PER_TILE = QP // NS

SECT = 128
NSEC = 2
IDX_PER_SECT = SECT * HIST_P
MAXC = SECT * HIST
CCAP = MAXC + 256
DUMMY = SECT
SENT = 1 << 20

CIDX = 224
NBUF = 2

_UNPACK_PERM = np.zeros((D,), dtype=np.int32)
for _c in range(D // 32):
    for _j in range(16):
        _UNPACK_PERM[32 * _c + 2 * _j] = 32 * _c + _j
        _UNPACK_PERM[32 * _c + 2 * _j + 1] = 32 * _c + 16 + _j

_mesh = plsc.VectorSubcoreMesh(
    core_axis_name="c", subcore_axis_name="s", num_cores=NC, num_subcores=NS)


@functools.partial(
    pl.kernel,
    out_type=[jax.ShapeDtypeStruct((B, D), jnp.float32)] * 12,
    mesh=_mesh,
    scratch_types=[
        pltpu.VMEM((IDX_PER_SECT,), jnp.int32),
        pltpu.VMEM((CCAP,), jnp.int32),
        pltpu.VMEM((CCAP,), jnp.int32),
        pltpu.VMEM((NBUF, CIDX, DP), jnp.bfloat16),
        pltpu.VMEM((DUMMY + 8, D), jnp.float32),
        pltpu.VMEM((4, BPW), jnp.int32),
        pltpu.VMEM((2, BPW, D), jnp.float32),
        pltpu.VMEM_SHARED((QP, DP), jnp.bfloat16),
        pltpu.SemaphoreType.DMA,
        pltpu.SemaphoreType.DMA,
        pltpu.SemaphoreType.DMA,
    ],
    compiler_params=pltpu.CompilerParams(
        use_tc_tiling_on_sc=False, needs_layout_passes=False),
)
def _sc_embed(uidx, aidx, utidx, iidx, cidx_in, itidx,
              uid_tab, uage_tab, text_tab, iid_tab, icate_tab,
              out_uid, out_uage, out_iid, out_icate,
              out_utx0, out_utx1, out_utx2, out_utx3,
              out_itx0, out_itx1, out_itx2, out_itx3,
              idxraw_v, cidx_v, cslot_v, bufs_v, pool_v, fidx_v, frows_v,
              shared_tab, sem0, sem1, semf):
    sems = (sem0, sem1)
    c = lax.axis_index("c")
    s = lax.axis_index("s")
    wid = s * NC + c
    fbase = wid * BPW

    field_in = ((uidx, uid_tab), (aidx, uage_tab),
                (iidx, iid_tab), (cidx_in, icate_tab))
    field_out = (out_uid, out_uage, out_iid, out_icate)

    def field_fire(f):
        pltpu.sync_copy(field_in[f][0].at[pl.ds(fbase, BPW)], fidx_v.at[f])
        pltpu.async_copy(
            field_in[f][1].at[fidx_v.at[f]], frows_v.at[f % 2], semf)

    def field_drain(f):
        pltpu.make_async_copy(
            field_in[f][1].at[fidx_v.at[f]], frows_v.at[f % 2], semf).wait()
        pltpu.sync_copy(
            frows_v.at[f % 2], field_out[f].at[pl.ds(fbase, BPW), :])

    field_fire(0)
    field_fire(1)

    tower_in = (utidx, itidx)
    tower_out = ((out_utx0, out_utx1, out_utx2, out_utx3),
                 (out_itx0, out_itx1, out_itx2, out_itx3))

    def section(qbase, tidx_hbm, out0, out1, hb):
        def zrow(i, carry):
            for cc in range(NCH):
                pool_v[i, pl.ds(cc * L, L)] = jnp.zeros((L,), jnp.float32)
            return carry

        lax.fori_loop(0, DUMMY + 1, zrow, 0, unroll=4)

        pltpu.sync_copy(
            tidx_hbm.at[pl.ds(s * 2 * IDX_PER_SECT + hb * IDX_PER_SECT,
                              IDX_PER_SECT)], idxraw_v)

        def grp(g, off):
            for cc in range(HIST_P // L):
                v = idxraw_v[pl.ds((g * (HIST_P // L) + cc) * L, L)]
                m = (v >= qbase) & (v < qbase + QL)
                lv = v - qbase
                plsc.store_compressed(
                    cidx_v.at[pl.ds(off, L)],
                    lax.shift_right_logical(lv, 1), mask=m)
                plsc.store_compressed(
                    cslot_v.at[pl.ds(off, L)],
                    jnp.full((L,), 2 * g, jnp.int32) + (lv & 1), mask=m)
                off = off + plsc.all_reduce_population_count(m)[0]
            return off

        total = lax.fori_loop(0, SECT, grp, 0, unroll=2)

        def fill(i, carry):
            cidx_v[pl.ds(total + i * L, L)] = jnp.zeros((L,), jnp.int32)
            cslot_v[pl.ds(total + i * L, L)] = jnp.full(
                (L,), 2 * DUMMY, jnp.int32)
            return carry

        lax.fori_loop(0, CIDX // L + 1, fill, 0)
        nchunk = (total + CIDX - 1) // CIDX

        def fire(k, n):
            pltpu.async_copy(
                shared_tab.at[cidx_v.at[pl.ds(k * CIDX, CIDX)]],
                bufs_v.at[n], sems[n])

        def wait(k, n):
            pltpu.make_async_copy(
                shared_tab.at[cidx_v.at[pl.ds(k * CIDX, CIDX)]],
                bufs_v.at[n], sems[n]).wait()

        def accumulate(k, n):
            rb = bufs_v.at[n]

            def row(r, carry):
                enc = cslot_v[pl.ds(k * CIDX + r, L)][0]
                slot = lax.shift_right_logical(enc, 1)
                colbase = (enc & 1) * D
                for h in range(NCH // 2):
                    lo, hi = plsc.unpack(
                        rb[r, pl.ds(colbase + h * 2 * L, 2 * L)],
                        format=plsc.PackFormat.INTERLEAVED)
                    plsc.addupdate(
                        pool_v.at[slot, pl.ds(2 * h * L, L)], lo)
                    plsc.addupdate(
                        pool_v.at[slot, pl.ds((2 * h + 1) * L, L)], hi)
                return carry

            lax.fori_loop(0, CIDX, row, 0, unroll=4)

        for n in range(NBUF):
            @pl.when(n < nchunk)
            def _():
                fire(n, n)

        def body(kk, carry):
            for n in range(NBUF):
                k = kk * NBUF + n

                @pl.when(k < nchunk)
                def _():
                    wait(k, n)
                    accumulate(k, n)

                @pl.when(k + NBUF < nchunk)
                def _():
                    fire(k + NBUF, n)
            return carry

        lax.fori_loop(0, (nchunk + NBUF - 1) // NBUF, body, 0)

        rowbase = s * 2 * SECT + hb * SECT

        @pl.when(c == 0)
        def _():
            pltpu.sync_copy(pool_v.at[pl.ds(0, SECT)],
                            out0.at[pl.ds(rowbase, SECT), :])

        @pl.when(c == 1)
        def _():
            pltpu.sync_copy(pool_v.at[pl.ds(0, SECT)],
                            out1.at[pl.ds(rowbase, SECT), :])

    for pss in range(2):
        plsc.subcore_barrier()
        qbase = (2 * pss) * QL + c * QL
        pltpu.sync_copy(
            text_tab.at[pl.ds(pss * 2 * QP + c * QP + s * STAGE_PER_TILE,
                              STAGE_PER_TILE), :],
            shared_tab.at[pl.ds(s * STAGE_PER_TILE, STAGE_PER_TILE), :])
        plsc.subcore_barrier()

        for tw in range(2):
            outs = tower_out[tw]
            for hb in range(2):
                section(qbase, tower_in[tw],
                        outs[2 * pss], outs[2 * pss + 1], hb)

        if pss == 0:
            field_drain(0)
            field_fire(2)
            field_drain(1)
            field_fire(3)

    field_drain(2)
    field_drain(3)


def _tc_body(euid, euage, eutx0, eutx1, eutx2, eutx3,
             eiid, eicate, eitx0, eitx1, eitx2, eitx3,
             uw1, ub1, uw2, ub2, iw1, ib1, iw2, ib2, out):
    f32 = jnp.float32

    def tower(e1, e2, e3, w1, b1, w2, b2):
        h = (jnp.dot(e1, w1[0:D], preferred_element_type=f32)
             + jnp.dot(e2, w1[D:2 * D], preferred_element_type=f32)
             + jnp.dot(e3, w1[2 * D:3 * D], preferred_element_type=f32)
             + b1[...])
        h = jnp.maximum(h, 0.0)
        o = jnp.dot(h, w2[...], preferred_element_type=f32) + b2[...]
        return jnp.maximum(o, 0.0)

    eutx = eutx0[...] + eutx1[...] + eutx2[...] + eutx3[...]
    eitx = eitx0[...] + eitx1[...] + eitx2[...] + eitx3[...]
    u = tower(euid[...], euage[...], eutx, uw1, ub1, uw2, ub2)
    it = tower(eiid[...], eicate[...], eitx, iw1, ib1, iw2, ib2)
    dot = jnp.sum(u * it)
    nu = jnp.sum(u * u)
    ni = jnp.sum(it * it)
    out[0, 0] = dot / (jnp.sqrt(nu) * jnp.sqrt(ni))


_tc_call = pl.pallas_call(
    _tc_body,
    out_shape=jax.ShapeDtypeStruct((1, 1), jnp.float32),
    out_specs=pl.BlockSpec(memory_space=pltpu.SMEM),
)


def kernel(user_id, user_age, user_text, item_id, item_cate, item_text,
           user_id_table, user_age_table, text_table, item_id_table,
           item_cate_table, u_w1, u_b1, u_w2, u_b2, i_w1, i_b1, i_w2, i_b2):
    uidx = user_id.reshape(B).astype(jnp.int32)
    aidx = user_age.reshape(B).astype(jnp.int32)
    iidx = item_id.reshape(B).astype(jnp.int32)
    cidx = item_cate.reshape(B).astype(jnp.int32)
    utp = jnp.pad(user_text.astype(jnp.int32),
                  ((0, 0), (0, HIST_P - HIST)),
                  constant_values=SENT).reshape(B * HIST_P)
    itp = jnp.pad(item_text.astype(jnp.int32),
                  ((0, 0), (0, HIST_P - HIST)),
                  constant_values=SENT).reshape(B * HIST_P)
    tt = jnp.pad((text_table * (1.0 / HIST)).astype(jnp.bfloat16)
                 [:, _UNPACK_PERM],
                 ((0, VT_PAD - VT), (0, 0))).reshape(VT_PAD // 2, DP)
    (euid, euage, eiid, eicate,
     eutx0, eutx1, eutx2, eutx3,
     eitx0, eitx1, eitx2, eitx3) = _sc_embed(
        uidx, aidx, utp, iidx, cidx, itp,
        user_id_table, user_age_table, tt, item_id_table, item_cate_table)
    score = _tc_call(
        euid, euage, eutx0, eutx1, eutx2, eutx3,
        eiid, eicate, eitx0, eitx1, eitx2, eitx3,
        u_w1, u_b1.reshape(1, H1), u_w2, u_b2.reshape(1, H2),
        i_w1, i_b1.reshape(1, H1), i_w2, i_b2.reshape(1, H2))
    return score.reshape(())


# speedup vs baseline: 2.2085x; 1.2842x over previous
import functools

import jax
import jax.numpy as jnp
import numpy as np
from jax import lax
from jax.experimental import pallas as pl
from jax.experimental.pallas import tpu as pltpu
from jax.experimental.pallas import tpu_sc as plsc

B = 4096
D = 64
HIST = 50
HIST_P = 64
H1, H2 = 64, 32
NC, NS, L = 2, 16, 16
NW = NC * NS
BPW = B // NW
NCH = D // L
DP = 2 * D

VT = 100001
QL = 25088
VT_PAD = 4 * QL
STAGE_PER_TILE = QL // NS
PS = 136

SECT = 128
NSEC = 2
IDX_PER_SECT = SECT * HIST_P
MAXC = SECT * HIST
CCAP = MAXC + 256
DUMMY = SECT
SENT = 1 << 20

CIDX = 128
NBUF = 2

_UNPACK_PERM = np.zeros((D,), dtype=np.int32)
for _c in range(D // 32):
    for _j in range(16):
        _UNPACK_PERM[32 * _c + 2 * _j] = 32 * _c + _j
        _UNPACK_PERM[32 * _c + 2 * _j + 1] = 32 * _c + 16 + _j

_mesh = plsc.VectorSubcoreMesh(
    core_axis_name="c", subcore_axis_name="s", num_cores=NC, num_subcores=NS)


@functools.partial(
    pl.kernel,
    out_type=[jax.ShapeDtypeStruct((B, D), jnp.float32)] * 12,
    mesh=_mesh,
    scratch_types=[
        pltpu.VMEM((IDX_PER_SECT,), jnp.int32),
        pltpu.VMEM((CCAP,), jnp.int32),
        pltpu.VMEM((CCAP,), jnp.int32),
        pltpu.VMEM((NBUF, CIDX, D), jnp.bfloat16),
        pltpu.VMEM((NBUF, CIDX, D), jnp.float32),
        pltpu.VMEM((NBUF, CIDX), jnp.int32),
        pltpu.VMEM((4, BPW), jnp.int32),
        pltpu.VMEM((2, BPW, D), jnp.float32),
        pltpu.VMEM_SHARED((QL, D), jnp.bfloat16),
        pltpu.VMEM_SHARED((NS * PS, D), jnp.float32),
        pltpu.SemaphoreType.DMA,
        pltpu.SemaphoreType.DMA,
        pltpu.SemaphoreType.DMA,
    ],
    compiler_params=pltpu.CompilerParams(
        use_tc_tiling_on_sc=False, needs_layout_passes=False),
)
def _sc_embed(uidx, aidx, utidx, iidx, cidx_in, itidx,
              uid_tab, uage_tab, text_tab, iid_tab, icate_tab,
              out_uid, out_uage, out_iid, out_icate,
              out_utx0, out_utx1, out_utx2, out_utx3,
              out_itx0, out_itx1, out_itx2, out_itx3,
              idxraw_v, cidx_v, cslot_v, bufs_v, fconv_v, sidx_v,
              fidx_v, frows_v, shared_tab, pool_sh, sem0, sem1, semf):
    sems = (sem0, sem1)
    c = lax.axis_index("c")
    s = lax.axis_index("s")
    wid = s * NC + c
    fbase = wid * BPW

    field_in = ((uidx, uid_tab), (aidx, uage_tab),
                (iidx, iid_tab), (cidx_in, icate_tab))
    field_out = (out_uid, out_uage, out_iid, out_icate)

    def field_fire(f):
        pltpu.sync_copy(field_in[f][0].at[pl.ds(fbase, BPW)], fidx_v.at[f])
        pltpu.async_copy(
            field_in[f][1].at[fidx_v.at[f]], frows_v.at[f % 2], semf)

    def field_drain(f):
        pltpu.make_async_copy(
            field_in[f][1].at[fidx_v.at[f]], frows_v.at[f % 2], semf).wait()
        pltpu.sync_copy(
            frows_v.at[f % 2], field_out[f].at[pl.ds(fbase, BPW), :])

    field_fire(0)
    field_fire(1)

    tower_in = (utidx, itidx)
    tower_out = ((out_utx0, out_utx1, out_utx2, out_utx3),
                 (out_itx0, out_itx1, out_itx2, out_itx3))

    def section(qbase, tidx_hbm, out0, out1, hb):
        def zrow(i, carry):
            for cc in range(NCH):
                fconv_v[0, i, pl.ds(cc * L, L)] = jnp.zeros((L,), jnp.float32)
            return carry

        lax.fori_loop(0, CIDX, zrow, 0, unroll=4)
        pltpu.sync_copy(fconv_v.at[0], pool_sh.at[pl.ds(s * PS, CIDX)])
        pltpu.sync_copy(fconv_v.at[0].at[pl.ds(0, PS - CIDX)],
                        pool_sh.at[pl.ds(s * PS + CIDX, PS - CIDX)])

        pltpu.sync_copy(
            tidx_hbm.at[pl.ds(s * 2 * IDX_PER_SECT + hb * IDX_PER_SECT,
                              IDX_PER_SECT)], idxraw_v)

        sbase = s * PS

        def grp(g, off):
            for cc in range(HIST_P // L):
                v = idxraw_v[pl.ds((g * (HIST_P // L) + cc) * L, L)]
                m = (v >= qbase) & (v < qbase + QL)
                plsc.store_compressed(
                    cidx_v.at[pl.ds(off, L)], v - qbase, mask=m)
                plsc.store_compressed(
                    cslot_v.at[pl.ds(off, L)],
                    jnp.full((L,), sbase + g, jnp.int32), mask=m)
                off = off + plsc.all_reduce_population_count(m)[0]
            return off

        total = lax.fori_loop(0, SECT, grp, 0)

        def fill(i, carry):
            cidx_v[pl.ds(total + i * L, L)] = jnp.zeros((L,), jnp.int32)
            cslot_v[pl.ds(total + i * L, L)] = jnp.full(
                (L,), sbase + DUMMY, jnp.int32)
            return carry

        lax.fori_loop(0, CIDX // L + 1, fill, 0)
        nchunk = (total + CIDX - 1) // CIDX

        def fire(k, n):
            pltpu.async_copy(
                shared_tab.at[cidx_v.at[pl.ds(k * CIDX, CIDX)]],
                bufs_v.at[n], sems[n])

        def wait(k, n):
            pltpu.make_async_copy(
                shared_tab.at[cidx_v.at[pl.ds(k * CIDX, CIDX)]],
                bufs_v.at[n], sems[n]).wait()

        def pool_chunk(k, n):
            rb = bufs_v.at[n]
            fc = fconv_v.at[n]

            def row(r, carry):
                for h in range(NCH // 2):
                    lo, hi = plsc.unpack(
                        rb[r, pl.ds(h * 2 * L, 2 * L)],
                        format=plsc.PackFormat.INTERLEAVED)
                    fc[r, pl.ds(2 * h * L, L)] = lo
                    fc[r, pl.ds((2 * h + 1) * L, L)] = hi
                return carry

            lax.fori_loop(0, CIDX, row, 0, unroll=4)

            def scp(i, carry):
                sidx_v[n, pl.ds(i * L, L)] = (
                    cslot_v[pl.ds(k * CIDX + i * L, L)])
                return carry

            lax.fori_loop(0, CIDX // L, scp, 0)
            pltpu.sync_copy(fc, pool_sh.at[sidx_v.at[n]], add=True)

        for n in range(NBUF):
            @pl.when(n < nchunk)
            def _():
                fire(n, n)

        def body(kk, carry):
            for n in range(NBUF):
                k = kk * NBUF + n

                @pl.when(k < nchunk)
                def _():
                    wait(k, n)
                    pool_chunk(k, n)

                @pl.when(k + NBUF < nchunk)
                def _():
                    fire(k + NBUF, n)
            return carry

        lax.fori_loop(0, (nchunk + NBUF - 1) // NBUF, body, 0)

        rowbase = s * 2 * SECT + hb * SECT

        @pl.when(c == 0)
        def _():
            pltpu.sync_copy(pool_sh.at[pl.ds(s * PS, SECT)],
                            out0.at[pl.ds(rowbase, SECT), :])

        @pl.when(c == 1)
        def _():
            pltpu.sync_copy(pool_sh.at[pl.ds(s * PS, SECT)],
                            out1.at[pl.ds(rowbase, SECT), :])

    for pss in range(2):
        plsc.subcore_barrier()
        qbase = (2 * pss) * QL + c * QL
        pltpu.sync_copy(
            text_tab.at[pl.ds(qbase + s * STAGE_PER_TILE,
                              STAGE_PER_TILE), :],
            shared_tab.at[pl.ds(s * STAGE_PER_TILE, STAGE_PER_TILE), :])
        plsc.subcore_barrier()

        for tw in range(2):
            outs = tower_out[tw]
            for hb in range(2):
                section(qbase, tower_in[tw],
                        outs[2 * pss], outs[2 * pss + 1], hb)

        if pss == 0:
            field_drain(0)
            field_fire(2)
            field_drain(1)
            field_fire(3)

    field_drain(2)
    field_drain(3)


def _tc_body(euid, euage, eutx0, eutx1, eutx2, eutx3,
             eiid, eicate, eitx0, eitx1, eitx2, eitx3,
             uw1, ub1, uw2, ub2, iw1, ib1, iw2, ib2, out):
    f32 = jnp.float32

    def tower(e1, e2, e3, w1, b1, w2, b2):
        h = (jnp.dot(e1, w1[0:D], preferred_element_type=f32)
             + jnp.dot(e2, w1[D:2 * D], preferred_element_type=f32)
             + jnp.dot(e3, w1[2 * D:3 * D], preferred_element_type=f32)
             + b1[...])
        h = jnp.maximum(h, 0.0)
        o = jnp.dot(h, w2[...], preferred_element_type=f32) + b2[...]
        return jnp.maximum(o, 0.0)

    eutx = eutx0[...] + eutx1[...] + eutx2[...] + eutx3[...]
    eitx = eitx0[...] + eitx1[...] + eitx2[...] + eitx3[...]
    u = tower(euid[...], euage[...], eutx, uw1, ub1, uw2, ub2)
    it = tower(eiid[...], eicate[...], eitx, iw1, ib1, iw2, ib2)
    dot = jnp.sum(u * it)
    nu = jnp.sum(u * u)
    ni = jnp.sum(it * it)
    out[0, 0] = dot / (jnp.sqrt(nu) * jnp.sqrt(ni))


_tc_call = pl.pallas_call(
    _tc_body,
    out_shape=jax.ShapeDtypeStruct((1, 1), jnp.float32),
    out_specs=pl.BlockSpec(memory_space=pltpu.SMEM),
)


def kernel(user_id, user_age, user_text, item_id, item_cate, item_text,
           user_id_table, user_age_table, text_table, item_id_table,
           item_cate_table, u_w1, u_b1, u_w2, u_b2, i_w1, i_b1, i_w2, i_b2):
    uidx = user_id.reshape(B).astype(jnp.int32)
    aidx = user_age.reshape(B).astype(jnp.int32)
    iidx = item_id.reshape(B).astype(jnp.int32)
    cidx = item_cate.reshape(B).astype(jnp.int32)
    utp = jnp.pad(user_text.astype(jnp.int32),
                  ((0, 0), (0, HIST_P - HIST)),
                  constant_values=SENT).reshape(B * HIST_P)
    itp = jnp.pad(item_text.astype(jnp.int32),
                  ((0, 0), (0, HIST_P - HIST)),
                  constant_values=SENT).reshape(B * HIST_P)
    tt = jnp.pad((text_table * (1.0 / HIST)).astype(jnp.bfloat16)
                 [:, _UNPACK_PERM], ((0, VT_PAD - VT), (0, 0)))
    (euid, euage, eiid, eicate,
     eutx0, eutx1, eutx2, eutx3,
     eitx0, eitx1, eitx2, eitx3) = _sc_embed(
        uidx, aidx, utp, iidx, cidx, itp,
        user_id_table, user_age_table, tt, item_id_table, item_cate_table)
    score = _tc_call(
        euid, euage, eutx0, eutx1, eutx2, eutx3,
        eiid, eicate, eitx0, eitx1, eitx2, eitx3,
        u_w1, u_b1.reshape(1, H1), u_w2, u_b2.reshape(1, H2),
        i_w1, i_b1.reshape(1, H1), i_w2, i_b2.reshape(1, H2))
    return score.reshape(())
